# baseline (speedup 1.0000x reference)
import math

import jax
import jax.numpy as jnp
import numpy as np
from jax import lax
from jax.experimental import pallas as pl
from jax.experimental.pallas import tpu as pltpu
from jax.experimental.pallas import tpu_sc as plsc

VOCAB = 1000000
D = 64
B = 4096
L = 200
N = B * L
NW = 32
NPW = N // NW
C = 80
NCH = NPW // C
VPR = D // 16
PEROWS = L + C


def _make_pe():
    pe = np.zeros((L, D), dtype=np.float32)
    position = np.arange(0.0, L, dtype=np.float64)[:, None]
    div_term = np.exp(
        np.arange(0.0, D, 2, dtype=np.float64) * -(math.log(10000.0) / D))
    pe[:, 0::2] = np.sin(position * div_term)
    pe[:, 1::2] = np.cos(position * div_term)
    return np.concatenate([pe, pe[:C]], axis=0)


_PE_EXT = _make_pe()

_SCALE = math.sqrt(D)


def _emb_body(x_hbm, pe_hbm, lut_hbm, out_hbm, idx_v, pe_v, rows_v, sem):
    cid = lax.axis_index("c")
    sid = lax.axis_index("s")
    wid = sid * 2 + cid

    pltpu.sync_copy(x_hbm.at[wid], idx_v)
    pltpu.sync_copy(pe_hbm, pe_v)

    def chunk(c, carry):
        ph = lax.rem(c * C, L)
        pltpu.async_copy(lut_hbm.at[idx_v.at[c]], rows_v, sem).wait()

        def row(j, carry2):
            for d in range(VPR):
                sl = pl.ds(d * 16, 16)
                rows_v[j, sl] = rows_v[j, sl] * _SCALE + pe_v[ph + j, sl]
            return carry2

        lax.fori_loop(0, C, row, 0)
        base = wid * NPW + c * C
        pltpu.sync_copy(rows_v, out_hbm.at[pl.ds(base, C)])
        return carry

    lax.fori_loop(0, NCH, chunk, 0)


_emb_call = pl.kernel(
    _emb_body,
    out_type=jax.ShapeDtypeStruct((N, D), jnp.float32),
    mesh=plsc.VectorSubcoreMesh(core_axis_name="c", subcore_axis_name="s"),
    compiler_params=pltpu.CompilerParams(use_tc_tiling_on_sc=False),
    scratch_types=[
        pltpu.VMEM((NCH, C), jnp.int32),
        pltpu.VMEM((PEROWS, D), jnp.float32),
        pltpu.VMEM((C, D), jnp.float32),
        pltpu.SemaphoreType.DMA,
    ],
)


def kernel(x, lut):
    xr = x.reshape(NW, NCH, C).astype(jnp.int32)
    pe = jnp.asarray(_PE_EXT)
    out = _emb_call(xr, pe, lut)
    return out.reshape(B, L, D)


# speedup vs baseline: 1.2192x; 1.2192x over previous
import math

import jax
import jax.numpy as jnp
import numpy as np
from jax import lax
from jax.experimental import pallas as pl
from jax.experimental.pallas import tpu as pltpu
from jax.experimental.pallas import tpu_sc as plsc

VOCAB = 1000000
D = 64
B = 4096
L = 200
N = B * L
NW = 32
NPW = N // NW
C = 80
NCH = NPW // C
VPR = D // 16
PEROWS = L + C
NB = 4
NR = NCH // NB


def _make_pe():
    pe = np.zeros((L, D), dtype=np.float32)
    position = np.arange(0.0, L, dtype=np.float64)[:, None]
    div_term = np.exp(
        np.arange(0.0, D, 2, dtype=np.float64) * -(math.log(10000.0) / D))
    pe[:, 0::2] = np.sin(position * div_term)
    pe[:, 1::2] = np.cos(position * div_term)
    return np.concatenate([pe, pe[:C]], axis=0)


_PE_EXT = _make_pe()

_SCALE = math.sqrt(D)


def _emb_body(x_hbm, pe_hbm, lut_hbm, out_hbm,
              idx_v, pe_v, r0, r1, r2, r3,
              g0, g1, g2, g3, o0, o1, o2, o3):
    rows = [r0, r1, r2, r3]
    gsem = [g0, g1, g2, g3]
    osem = [o0, o1, o2, o3]

    cid = lax.axis_index("c")
    sid = lax.axis_index("s")
    wid = sid * 2 + cid
    obase = wid * NPW

    pltpu.sync_copy(x_hbm.at[wid], idx_v)
    pltpu.sync_copy(pe_hbm, pe_v)

    pltpu.async_copy(lut_hbm.at[idx_v.at[0]], rows[0], gsem[0])
    pltpu.async_copy(lut_hbm.at[idx_v.at[1]], rows[1], gsem[1])

    def rnd(g, carry):
        for b in range(NB):
            c = g * NB + b
            b2 = (b + 2) % NB

            if b < 2:
                @pl.when(g >= 1)
                def _():
                    pltpu.make_async_copy(
                        rows[b2], out_hbm.at[pl.ds(obase, C)], osem[b2]
                    ).wait()
                pltpu.async_copy(
                    lut_hbm.at[idx_v.at[c + 2]], rows[b2], gsem[b2])
            else:
                pltpu.make_async_copy(
                    rows[b2], out_hbm.at[pl.ds(obase, C)], osem[b2]
                ).wait()

                @pl.when(g < NR - 1)
                def _():
                    pltpu.async_copy(
                        lut_hbm.at[idx_v.at[c + 2]], rows[b2], gsem[b2])

            pltpu.make_async_copy(
                lut_hbm.at[idx_v.at[c]], rows[b], gsem[b]).wait()

            ph = lax.rem(c * C, L)

            def row(j, carry2):
                for d in range(VPR):
                    sl = pl.ds(d * 16, 16)
                    rows[b][j, sl] = rows[b][j, sl] * _SCALE + pe_v[ph + j, sl]
                return carry2

            lax.fori_loop(0, C, row, 0)

            pltpu.async_copy(
                rows[b], out_hbm.at[pl.ds(obase + c * C, C)], osem[b])
        return carry

    lax.fori_loop(0, NR, rnd, 0)

    for b in (2, 3):
        pltpu.make_async_copy(
            rows[b], out_hbm.at[pl.ds(obase, C)], osem[b]).wait()


_emb_call = pl.kernel(
    _emb_body,
    out_type=jax.ShapeDtypeStruct((N, D), jnp.float32),
    mesh=plsc.VectorSubcoreMesh(core_axis_name="c", subcore_axis_name="s"),
    compiler_params=pltpu.CompilerParams(use_tc_tiling_on_sc=False),
    scratch_types=(
        [pltpu.VMEM((NCH, C), jnp.int32),
         pltpu.VMEM((PEROWS, D), jnp.float32)]
        + [pltpu.VMEM((C, D), jnp.float32) for _ in range(NB)]
        + [pltpu.SemaphoreType.DMA for _ in range(2 * NB)]
    ),
)


def kernel(x, lut):
    xr = x.reshape(NW, NCH, C).astype(jnp.int32)
    pe = jnp.asarray(_PE_EXT)
    out = _emb_call(xr, pe, lut)
    return out.reshape(B, L, D)


# speedup vs baseline: 1.4208x; 1.1654x over previous
import math

import jax
import jax.numpy as jnp
import numpy as np
from jax import lax
from jax.experimental import pallas as pl
from jax.experimental.pallas import tpu as pltpu
from jax.experimental.pallas import tpu_sc as plsc

VOCAB = 1000000
D = 64
B = 4096
L = 200
N = B * L
NW = 32
NPW = N // NW
C = 128
NCH = NPW // C
NB16 = C // 16
PEROWS = L + C
PELINES = (PEROWS // 2 + 7) // 8 * 8


def _make_pe():
    pe = np.zeros((PEROWS, D), dtype=np.float32)
    position = np.arange(0.0, PEROWS, dtype=np.float64)[:, None] % L
    div_term = np.exp(
        np.arange(0.0, D, 2, dtype=np.float64) * -(math.log(10000.0) / D))
    pe[:, 0::2] = np.sin(position * div_term)
    pe[:, 1::2] = np.cos(position * div_term)
    out = np.zeros((PELINES, 2 * D), dtype=np.float32)
    out.reshape(-1)[: PEROWS * D] = pe.reshape(-1)
    return out


_PE_PACKED = _make_pe()

_SCALE = math.sqrt(D)


def _emb_body(x_hbm, pe_hbm, lut_hbm, out_hbm,
              idx_v, pe_v, r0, r1, o0, o1, x0, x1,
              g0, g1, s0, s1):
    rows = [r0, r1]
    outc = [o0, o1]
    gidx = [x0, x1]
    gsem = [g0, g1]
    osem = [s0, s1]

    cid = lax.axis_index("c")
    sid = lax.axis_index("s")
    wid = sid * 2 + cid
    obase = wid * NPW

    pltpu.sync_copy(x_hbm.at[wid], idx_v)
    pltpu.sync_copy(pe_hbm, pe_v)

    def make_gidx(cc, slot):
        def blk(b16, carry):
            sl = pl.ds(b16 * 16, 16)
            gidx[slot][sl] = lax.shift_right_logical(idx_v[cc, sl], 1)
            return carry
        lax.fori_loop(0, NB16, blk, 0)

    make_gidx(0, 0)
    pltpu.async_copy(lut_hbm.at[gidx[0]], rows[0], gsem[0])

    def chunk(c, carry):
        r = lax.rem(c, 2)

        @pl.when(r == 0)
        def _():
            pltpu.make_async_copy(lut_hbm.at[gidx[0]], rows[0], gsem[0]).wait()

        @pl.when(r == 1)
        def _():
            pltpu.make_async_copy(lut_hbm.at[gidx[1]], rows[1], gsem[1]).wait()

        @pl.when(c < NCH - 1)
        def _():
            @pl.when(r == 0)
            def _():
                make_gidx(c + 1, 1)
                pltpu.async_copy(lut_hbm.at[gidx[1]], rows[1], gsem[1])

            @pl.when(r == 1)
            def _():
                make_gidx(c + 1, 0)
                pltpu.async_copy(lut_hbm.at[gidx[0]], rows[0], gsem[0])

        @pl.when(c >= 2)
        def _():
            @pl.when(r == 0)
            def _():
                pltpu.make_async_copy(
                    outc[0], out_hbm.at[pl.ds(obase, C)], osem[0]).wait()

            @pl.when(r == 1)
            def _():
                pltpu.make_async_copy(
                    outc[1], out_hbm.at[pl.ds(obase, C)], osem[1]).wait()

        ph = lax.rem(c * C, L)
        phh = lax.shift_right_logical(ph, 1)

        def compute(rv, ov):
            def blk(b16, carry2):
                jb = b16 * 16
                sl = pl.ds(jb, 16)
                off16 = (idx_v[c, sl] & 1) * D
                for jj in range(16):
                    j = jb + jj
                    off = off16[jj]
                    prow = phh + b16 * 8 + (jj >> 1)
                    pc0 = (jj & 1) * D
                    for d in range(D // 16):
                        psl = pl.ds(pc0 + d * 16, 16)
                        ov[j, pl.ds(d * 16, 16)] = (
                            rv[j, pl.ds(off + d * 16, 16)] * _SCALE
                            + pe_v[prow, psl])
                return carry2
            lax.fori_loop(0, NB16, blk, 0)

        @pl.when(r == 0)
        def _():
            compute(rows[0], outc[0])
            pltpu.async_copy(
                outc[0], out_hbm.at[pl.ds(obase + c * C, C)], osem[0])

        @pl.when(r == 1)
        def _():
            compute(rows[1], outc[1])
            pltpu.async_copy(
                outc[1], out_hbm.at[pl.ds(obase + c * C, C)], osem[1])

        return carry

    lax.fori_loop(0, NCH, chunk, 0)

    for b in range(2):
        pltpu.make_async_copy(
            outc[b], out_hbm.at[pl.ds(obase, C)], osem[b]).wait()


_emb_call = pl.kernel(
    _emb_body,
    out_type=jax.ShapeDtypeStruct((N, D), jnp.float32),
    mesh=plsc.VectorSubcoreMesh(core_axis_name="c", subcore_axis_name="s"),
    scratch_types=(
        [pltpu.VMEM((NCH, C), jnp.int32),
         pltpu.VMEM((PELINES, 2 * D), jnp.float32)]
        + [pltpu.VMEM((C, 2 * D), jnp.float32) for _ in range(2)]
        + [pltpu.VMEM((C, D), jnp.float32) for _ in range(2)]
        + [pltpu.VMEM((C,), jnp.int32) for _ in range(2)]
        + [pltpu.SemaphoreType.DMA for _ in range(4)]
    ),
)


def kernel(x, lut):
    xr = x.reshape(NW, NCH, C).astype(jnp.int32)
    lut2 = lut.reshape(VOCAB // 2, 2 * D)
    pe = jnp.asarray(_PE_PACKED)
    out = _emb_call(xr, pe, lut2)
    return out.reshape(B, L, D)
